# baseline (speedup 1.0000x reference)
import functools

import jax
import jax.numpy as jnp
from jax import lax
from jax.experimental import pallas as pl
from jax.experimental.pallas import tpu as pltpu
from jax.experimental.pallas import tpu_sc as plsc

B = 16384
E = 30

_info = plsc.get_sparse_core_info()
_NC, _NS = _info.num_cores, _info.num_subcores
_NW = _NC * _NS
_BW = B // _NW


def _make_gather_kernel():
    mesh = plsc.VectorSubcoreMesh(core_axis_name="c", subcore_axis_name="s")

    @functools.partial(
        pl.kernel,
        mesh=mesh,
        compiler_params=pltpu.CompilerParams(use_tc_tiling_on_sc=False),
        out_type=[jax.ShapeDtypeStruct((B, E), jnp.float32)] * 4,
        scratch_types=[
            pltpu.VMEM((_BW,), jnp.int32),
            pltpu.VMEM((_BW,), jnp.int32),
            pltpu.VMEM((_BW, E), jnp.float32),
            pltpu.VMEM((_BW, E), jnp.float32),
            pltpu.VMEM((_BW, E), jnp.float32),
            pltpu.VMEM((_BW, E), jnp.float32),
            pltpu.SemaphoreType.DMA,
            pltpu.SemaphoreType.DMA,
            pltpu.SemaphoreType.DMA,
            pltpu.SemaphoreType.DMA,
        ],
    )
    def gather_k(uidx_hbm, midx_hbm, uml_hbm, mml_hbm, umf_hbm, mmf_hbm,
                 o_uml, o_mml, o_umf, o_mmf,
                 uidx_v, midx_v, buml, bmml, bumf, bmmf, s0, s1, s2, s3):
        wid = lax.axis_index("s") * _NC + lax.axis_index("c")
        base = wid * _BW
        pltpu.sync_copy(uidx_hbm.at[pl.ds(base, _BW)], uidx_v)
        pltpu.sync_copy(midx_hbm.at[pl.ds(base, _BW)], midx_v)
        c0 = pltpu.async_copy(uml_hbm.at[uidx_v], buml, s0)
        c1 = pltpu.async_copy(mml_hbm.at[midx_v], bmml, s1)
        c2 = pltpu.async_copy(umf_hbm.at[uidx_v], bumf, s2)
        c3 = pltpu.async_copy(mmf_hbm.at[midx_v], bmmf, s3)
        c0.wait()
        c1.wait()
        c2.wait()
        c3.wait()
        pltpu.sync_copy(buml, o_uml.at[pl.ds(base, _BW)])
        pltpu.sync_copy(bmml, o_mml.at[pl.ds(base, _BW)])
        pltpu.sync_copy(bumf, o_umf.at[pl.ds(base, _BW)])
        pltpu.sync_copy(bmmf, o_mmf.at[pl.ds(base, _BW)])

    return gather_k


_gather = _make_gather_kernel()

_BLK = 1024
_NBLK = B // _BLK


def _mlp_body(ue, me, uf, mf, w0u, w0m, b0, w1t, b1, w2t, b2, am, af, ab,
              out_ref):
    x0 = (jnp.dot(ue[...], w0u[...], preferred_element_type=jnp.float32)
          + jnp.dot(me[...], w0m[...], preferred_element_type=jnp.float32)
          + b0[...])
    h0 = jnp.maximum(x0, 0.0)
    h1 = jnp.maximum(
        jnp.dot(h0, w1t[...], preferred_element_type=jnp.float32) + b1[...],
        0.0)
    h2 = jnp.maximum(
        jnp.dot(h1, w2t[...], preferred_element_type=jnp.float32) + b2[...],
        0.0)
    mfv = uf[...] * mf[...]
    logit = (jnp.dot(h2, am[...], preferred_element_type=jnp.float32)
             + jnp.dot(mfv, af[...], preferred_element_type=jnp.float32)
             + ab[0, 0])
    out_ref[...] = logit * 5.0


def _rep(shape):
    return pl.BlockSpec(shape, lambda i: (0, 0))


_mlp_call = pl.pallas_call(
    _mlp_body,
    grid=(_NBLK,),
    in_specs=[
        pl.BlockSpec((_BLK, E), lambda i: (i, 0)),
        pl.BlockSpec((_BLK, E), lambda i: (i, 0)),
        pl.BlockSpec((_BLK, E), lambda i: (i, 0)),
        pl.BlockSpec((_BLK, E), lambda i: (i, 0)),
        _rep((E, 32)),
        _rep((E, 32)),
        _rep((1, 32)),
        _rep((32, 16)),
        _rep((1, 16)),
        _rep((16, 8)),
        _rep((1, 8)),
        _rep((8, 1)),
        _rep((E, 1)),
        _rep((1, 1)),
    ],
    out_specs=pl.BlockSpec((_BLK, 1), lambda i: (i, 0)),
    out_shape=jax.ShapeDtypeStruct((B, 1), jnp.float32),
)


def kernel(train_data, user_emb_mlp, movie_emb_mlp, user_emb_mf, movie_emb_mf,
           fc_w0, fc_b0, fc_w1, fc_b1, fc_w2, fc_b2, aff_w, aff_b):
    uidx = train_data[:, 0]
    midx = train_data[:, 1]
    uml_g, mml_g, umf_g, mmf_g = _gather(
        uidx, midx, user_emb_mlp, movie_emb_mlp, user_emb_mf, movie_emb_mf)
    w0u = fc_w0[:, :E].T
    w0m = fc_w0[:, E:].T
    out = _mlp_call(
        uml_g, mml_g, umf_g, mmf_g,
        w0u, w0m, fc_b0.reshape(1, -1),
        fc_w1.T, fc_b1.reshape(1, -1),
        fc_w2.T, fc_b2.reshape(1, -1),
        aff_w[0, :8].reshape(-1, 1), aff_w[0, 8:].reshape(-1, 1),
        aff_b.reshape(1, 1),
    )
    return out.reshape(-1)


# speedup vs baseline: 1.0775x; 1.0775x over previous
import functools

import jax
import jax.numpy as jnp
from jax import lax
from jax.experimental import pallas as pl
from jax.experimental.pallas import tpu as pltpu
from jax.experimental.pallas import tpu_sc as plsc

B = 16384
E = 30
EP = 32
_XW = 4 * EP

_info = plsc.get_sparse_core_info()
_NC, _NS = _info.num_cores, _info.num_subcores
_NW = _NC * _NS
_BW = B // _NW


def _make_gather_kernel():
    mesh = plsc.VectorSubcoreMesh(core_axis_name="c", subcore_axis_name="s")

    @functools.partial(
        pl.kernel,
        mesh=mesh,
        compiler_params=pltpu.CompilerParams(use_tc_tiling_on_sc=False),
        out_type=jax.ShapeDtypeStruct((B, _XW), jnp.float32),
        scratch_types=[
            pltpu.VMEM((_BW,), jnp.int32),
            pltpu.VMEM((_BW,), jnp.int32),
            pltpu.VMEM((_BW, EP), jnp.float32),
            pltpu.VMEM((_BW, EP), jnp.float32),
            pltpu.VMEM((_BW, EP), jnp.float32),
            pltpu.VMEM((_BW, EP), jnp.float32),
            pltpu.SemaphoreType.DMA,
            pltpu.SemaphoreType.DMA,
            pltpu.SemaphoreType.DMA,
            pltpu.SemaphoreType.DMA,
        ],
    )
    def gather_k(uidx_hbm, midx_hbm, uml_hbm, mml_hbm, umf_hbm, mmf_hbm,
                 o_x,
                 uidx_v, midx_v, buml, bmml, bumf, bmmf, s0, s1, s2, s3):
        wid = lax.axis_index("s") * _NC + lax.axis_index("c")
        base = wid * _BW
        pltpu.sync_copy(uidx_hbm.at[pl.ds(base, _BW)], uidx_v)
        pltpu.sync_copy(midx_hbm.at[pl.ds(base, _BW)], midx_v)
        c0 = pltpu.async_copy(uml_hbm.at[uidx_v], buml, s0)
        c1 = pltpu.async_copy(mml_hbm.at[midx_v], bmml, s1)
        c2 = pltpu.async_copy(umf_hbm.at[uidx_v], bumf, s2)
        c3 = pltpu.async_copy(mmf_hbm.at[midx_v], bmmf, s3)
        c0.wait()
        pltpu.sync_copy(buml, o_x.at[pl.ds(base, _BW), pl.ds(0 * EP, EP)])
        c1.wait()
        pltpu.sync_copy(bmml, o_x.at[pl.ds(base, _BW), pl.ds(1 * EP, EP)])
        c2.wait()
        pltpu.sync_copy(bumf, o_x.at[pl.ds(base, _BW), pl.ds(2 * EP, EP)])
        c3.wait()
        pltpu.sync_copy(bmmf, o_x.at[pl.ds(base, _BW), pl.ds(3 * EP, EP)])

    return gather_k


_gather = _make_gather_kernel()

_BLK = 1024
_NBLK = B // _BLK


def _mlp_body(x, w0u, w0m, b0, w1t, b1, w2t, b2, am, af, ab, out_ref):
    ue = x[:, 0 * EP:0 * EP + E]
    me = x[:, 1 * EP:1 * EP + E]
    uf = x[:, 2 * EP:2 * EP + E]
    mf = x[:, 3 * EP:3 * EP + E]
    x0 = (jnp.dot(ue, w0u[...], preferred_element_type=jnp.float32)
          + jnp.dot(me, w0m[...], preferred_element_type=jnp.float32)
          + b0[...])
    h0 = jnp.maximum(x0, 0.0)
    h1 = jnp.maximum(
        jnp.dot(h0, w1t[...], preferred_element_type=jnp.float32) + b1[...],
        0.0)
    h2 = jnp.maximum(
        jnp.dot(h1, w2t[...], preferred_element_type=jnp.float32) + b2[...],
        0.0)
    mfv = uf * mf
    logit = (jnp.dot(h2, am[...], preferred_element_type=jnp.float32)
             + jnp.dot(mfv, af[...], preferred_element_type=jnp.float32)
             + ab[0, 0])
    out_ref[...] = logit * 5.0


def _rep(shape):
    return pl.BlockSpec(shape, lambda i: (0, 0))


_mlp_call = pl.pallas_call(
    _mlp_body,
    grid=(_NBLK,),
    in_specs=[
        pl.BlockSpec((_BLK, _XW), lambda i: (i, 0)),
        _rep((E, 32)),
        _rep((E, 32)),
        _rep((1, 32)),
        _rep((32, 16)),
        _rep((1, 16)),
        _rep((16, 8)),
        _rep((1, 8)),
        _rep((8, 1)),
        _rep((E, 1)),
        _rep((1, 1)),
    ],
    out_specs=pl.BlockSpec((_BLK, 1), lambda i: (i, 0)),
    out_shape=jax.ShapeDtypeStruct((B, 1), jnp.float32),
)


def kernel(train_data, user_emb_mlp, movie_emb_mlp, user_emb_mf, movie_emb_mf,
           fc_w0, fc_b0, fc_w1, fc_b1, fc_w2, fc_b2, aff_w, aff_b):
    uidx = train_data[:, 0]
    midx = train_data[:, 1]
    pad = ((0, 0), (0, EP - E))
    x_g = _gather(
        uidx, midx,
        jnp.pad(user_emb_mlp, pad), jnp.pad(movie_emb_mlp, pad),
        jnp.pad(user_emb_mf, pad), jnp.pad(movie_emb_mf, pad))
    w0u = fc_w0[:, :E].T
    w0m = fc_w0[:, E:].T
    out = _mlp_call(
        x_g,
        w0u, w0m, fc_b0.reshape(1, -1),
        fc_w1.T, fc_b1.reshape(1, -1),
        fc_w2.T, fc_b2.reshape(1, -1),
        aff_w[0, :8].reshape(-1, 1), aff_w[0, 8:].reshape(-1, 1),
        aff_b.reshape(1, 1),
    )
    return out.reshape(-1)


# speedup vs baseline: 1.5679x; 1.4551x over previous
import functools

import jax
import jax.numpy as jnp
from jax import lax
from jax.experimental import pallas as pl
from jax.experimental.pallas import tpu as pltpu
from jax.experimental.pallas import tpu_sc as plsc

B = 16384
E = 30
EP = 32
_XW = 4 * EP

_info = plsc.get_sparse_core_info()
_NC, _NS = _info.num_cores, _info.num_subcores
_NW = _NC * _NS
_BW = B // _NW


def _make_gather_kernel():
    mesh = plsc.VectorSubcoreMesh(core_axis_name="c", subcore_axis_name="s")

    @functools.partial(
        pl.kernel,
        mesh=mesh,
        compiler_params=pltpu.CompilerParams(use_tc_tiling_on_sc=False),
        out_type=[jax.ShapeDtypeStruct((B, 128), jnp.float32)] * 4,
        scratch_types=[
            pltpu.VMEM((_BW,), jnp.int32),
            pltpu.VMEM((_BW,), jnp.int32),
            pltpu.VMEM((_BW, 128), jnp.float32),
            pltpu.SemaphoreType.DMA,
        ],
    )
    def gather_k(uidx_hbm, midx_hbm, uml_hbm, mml_hbm, umf_hbm, mmf_hbm,
                 o0, o1, o2, o3, uidx_v, midx_v, buf, s0):
        wid = lax.axis_index("s") * _NC + lax.axis_index("c")
        base = wid * _BW
        pltpu.sync_copy(uidx_hbm.at[pl.ds(base, _BW)], uidx_v)
        pltpu.sync_copy(midx_hbm.at[pl.ds(base, _BW)], midx_v)
        for tab, idx_v, out in ((uml_hbm, uidx_v, o0), (mml_hbm, midx_v, o1),
                                (umf_hbm, uidx_v, o2), (mmf_hbm, midx_v, o3)):
            pltpu.async_copy(tab.at[idx_v], buf, s0).wait()
            pltpu.sync_copy(buf, out.at[pl.ds(base, _BW)])

    return gather_k


_gather = _make_gather_kernel()

_BLK = 1024
_NBLK = B // _BLK


def _mlp_body(xue, xme, xuf, xmf, w0u, w0m, b0, w1t, b1, w2t, b2, am, af, ab,
              out_ref):
    ue = xue[:, :E]
    me = xme[:, :E]
    uf = xuf[:, :E]
    mf = xmf[:, :E]
    x0 = (jnp.dot(ue, w0u[...], preferred_element_type=jnp.float32)
          + jnp.dot(me, w0m[...], preferred_element_type=jnp.float32)
          + b0[...])
    h0 = jnp.maximum(x0, 0.0)
    h1 = jnp.maximum(
        jnp.dot(h0, w1t[...], preferred_element_type=jnp.float32) + b1[...],
        0.0)
    h2 = jnp.maximum(
        jnp.dot(h1, w2t[...], preferred_element_type=jnp.float32) + b2[...],
        0.0)
    mfv = uf * mf
    logit = (jnp.dot(h2, am[...], preferred_element_type=jnp.float32)
             + jnp.dot(mfv, af[...], preferred_element_type=jnp.float32)
             + ab[0, 0])
    out_ref[...] = logit * 5.0


def _rep(shape):
    return pl.BlockSpec(shape, lambda i: (0, 0))


_mlp_call = pl.pallas_call(
    _mlp_body,
    grid=(_NBLK,),
    in_specs=[
        pl.BlockSpec((_BLK, 128), lambda i: (i, 0)),
        pl.BlockSpec((_BLK, 128), lambda i: (i, 0)),
        pl.BlockSpec((_BLK, 128), lambda i: (i, 0)),
        pl.BlockSpec((_BLK, 128), lambda i: (i, 0)),
        _rep((E, 32)),
        _rep((E, 32)),
        _rep((1, 32)),
        _rep((32, 16)),
        _rep((1, 16)),
        _rep((16, 8)),
        _rep((1, 8)),
        _rep((8, 1)),
        _rep((E, 1)),
        _rep((1, 1)),
    ],
    out_specs=pl.BlockSpec((_BLK, 1), lambda i: (i, 0)),
    out_shape=jax.ShapeDtypeStruct((B, 1), jnp.float32),
)


def kernel(train_data, user_emb_mlp, movie_emb_mlp, user_emb_mf, movie_emb_mf,
           fc_w0, fc_b0, fc_w1, fc_b1, fc_w2, fc_b2, aff_w, aff_b):
    uidx = train_data[:, 0]
    midx = train_data[:, 1]
    pad = ((0, 0), (0, 128 - E))
    g0, g1, g2, g3 = _gather(
        uidx, midx,
        jnp.pad(user_emb_mlp, pad), jnp.pad(movie_emb_mlp, pad),
        jnp.pad(user_emb_mf, pad), jnp.pad(movie_emb_mf, pad))
    w0u = fc_w0[:, :E].T
    w0m = fc_w0[:, E:].T
    out = _mlp_call(
        g0, g1, g2, g3,
        w0u, w0m, fc_b0.reshape(1, -1),
        fc_w1.T, fc_b1.reshape(1, -1),
        fc_w2.T, fc_b2.reshape(1, -1),
        aff_w[0, :8].reshape(-1, 1), aff_w[0, 8:].reshape(-1, 1),
        aff_b.reshape(1, 1),
    )
    return out.reshape(-1)


# speedup vs baseline: 1.5752x; 1.0047x over previous
import functools

import jax
import jax.numpy as jnp
from jax import lax
from jax.experimental import pallas as pl
from jax.experimental.pallas import tpu as pltpu
from jax.experimental.pallas import tpu_sc as plsc

B = 16384
E = 30
EP = 32
_XW = 4 * EP

_info = plsc.get_sparse_core_info()
_NC, _NS = _info.num_cores, _info.num_subcores
_NW = _NC * _NS
_BW = B // _NW


def _make_gather_kernel():
    mesh = plsc.VectorSubcoreMesh(core_axis_name="c", subcore_axis_name="s")

    @functools.partial(
        pl.kernel,
        mesh=mesh,
        compiler_params=pltpu.CompilerParams(use_tc_tiling_on_sc=False),
        out_type=[jax.ShapeDtypeStruct((B, 128), jnp.float32)] * 4,
        scratch_types=[
            pltpu.VMEM((_BW,), jnp.int32),
            pltpu.VMEM((_BW,), jnp.int32),
            pltpu.VMEM((_BW // 2, 128), jnp.float32),
            pltpu.VMEM((_BW // 2, 128), jnp.float32),
            pltpu.SemaphoreType.DMA,
            pltpu.SemaphoreType.DMA,
        ],
    )
    def gather_k(uidx_hbm, midx_hbm, uml_hbm, mml_hbm, umf_hbm, mmf_hbm,
                 o0, o1, o2, o3, uidx_v, midx_v, buf_a, buf_b, s0, s1):
        wid = lax.axis_index("s") * _NC + lax.axis_index("c")
        base = wid * _BW
        hw = _BW // 2
        pltpu.sync_copy(uidx_hbm.at[pl.ds(base, _BW)], uidx_v)
        pltpu.sync_copy(midx_hbm.at[pl.ds(base, _BW)], midx_v)
        steps = []
        for tab, idx_v, out in ((uml_hbm, uidx_v, o0), (mml_hbm, midx_v, o1),
                                (umf_hbm, uidx_v, o2), (mmf_hbm, midx_v, o3)):
            for h in range(2):
                steps.append((tab, idx_v.at[pl.ds(h * hw, hw)],
                              out.at[pl.ds(base + h * hw, hw)]))
        bufs = (buf_a, buf_b)
        sems = (s0, s1)
        copies = [None, None]
        for k, (tab, idx_v, dst) in enumerate(steps):
            p = k % 2
            if copies[p] is not None:
                copies[p].wait()
                pltpu.sync_copy(bufs[p], steps[k - 2][2])
            copies[p] = pltpu.async_copy(tab.at[idx_v], bufs[p], sems[p])
        for k in (len(steps) - 2, len(steps) - 1):
            p = k % 2
            copies[p].wait()
            pltpu.sync_copy(bufs[p], steps[k][2])

    return gather_k


_gather = _make_gather_kernel()

_BLK = 1024
_NBLK = B // _BLK


def _mlp_body(xue, xme, xuf, xmf, w0u, w0m, b0, w1t, b1, w2t, b2, am, af, ab,
              out_ref):
    ue = xue[:, :E]
    me = xme[:, :E]
    uf = xuf[:, :E]
    mf = xmf[:, :E]
    x0 = (jnp.dot(ue, w0u[...], preferred_element_type=jnp.float32)
          + jnp.dot(me, w0m[...], preferred_element_type=jnp.float32)
          + b0[...])
    h0 = jnp.maximum(x0, 0.0)
    h1 = jnp.maximum(
        jnp.dot(h0, w1t[...], preferred_element_type=jnp.float32) + b1[...],
        0.0)
    h2 = jnp.maximum(
        jnp.dot(h1, w2t[...], preferred_element_type=jnp.float32) + b2[...],
        0.0)
    mfv = uf * mf
    logit = (jnp.dot(h2, am[...], preferred_element_type=jnp.float32)
             + jnp.dot(mfv, af[...], preferred_element_type=jnp.float32)
             + ab[0, 0])
    out_ref[...] = logit * 5.0


def _rep(shape):
    return pl.BlockSpec(shape, lambda i: (0, 0))


_mlp_call = pl.pallas_call(
    _mlp_body,
    grid=(_NBLK,),
    in_specs=[
        pl.BlockSpec((_BLK, 128), lambda i: (i, 0)),
        pl.BlockSpec((_BLK, 128), lambda i: (i, 0)),
        pl.BlockSpec((_BLK, 128), lambda i: (i, 0)),
        pl.BlockSpec((_BLK, 128), lambda i: (i, 0)),
        _rep((E, 32)),
        _rep((E, 32)),
        _rep((1, 32)),
        _rep((32, 16)),
        _rep((1, 16)),
        _rep((16, 8)),
        _rep((1, 8)),
        _rep((8, 1)),
        _rep((E, 1)),
        _rep((1, 1)),
    ],
    out_specs=pl.BlockSpec((_BLK, 1), lambda i: (i, 0)),
    out_shape=jax.ShapeDtypeStruct((B, 1), jnp.float32),
)


def kernel(train_data, user_emb_mlp, movie_emb_mlp, user_emb_mf, movie_emb_mf,
           fc_w0, fc_b0, fc_w1, fc_b1, fc_w2, fc_b2, aff_w, aff_b):
    uidx = train_data[:, 0]
    midx = train_data[:, 1]
    pad = ((0, 0), (0, 128 - E))
    g0, g1, g2, g3 = _gather(
        uidx, midx,
        jnp.pad(user_emb_mlp, pad), jnp.pad(movie_emb_mlp, pad),
        jnp.pad(user_emb_mf, pad), jnp.pad(movie_emb_mf, pad))
    w0u = fc_w0[:, :E].T
    w0m = fc_w0[:, E:].T
    out = _mlp_call(
        g0, g1, g2, g3,
        w0u, w0m, fc_b0.reshape(1, -1),
        fc_w1.T, fc_b1.reshape(1, -1),
        fc_w2.T, fc_b2.reshape(1, -1),
        aff_w[0, :8].reshape(-1, 1), aff_w[0, 8:].reshape(-1, 1),
        aff_b.reshape(1, 1),
    )
    return out.reshape(-1)
